# Initial kernel scaffold; baseline (speedup 1.0000x reference)
import jax
import jax.numpy as jnp
from jax.experimental import pallas as pl


def kernel(x, edge_index, W_l1, b_l1, W_r1, W_l2, b_l2, W_r2):
    raise NotImplementedError("write your pallas kernel here")


# speedup vs baseline: 3.8151x; 3.8151x over previous
import functools

import jax
import jax.numpy as jnp
from jax import lax
from jax.experimental import pallas as pl
from jax.experimental.pallas import tpu as pltpu
from jax.experimental.pallas import tpu_sc as plsc

N = 10000
E = 320000
D = 128
DA = 144
NC, NS = 2, 16
NW = NC * NS
CHUNK = 128
CPW = -(-E // (NW * CHUNK))
EPAD = NW * CPW * CHUNK
NPAD = NW * 320
ROWS_PT = NPAD // NS


def _sc_agg_body(x_hbm, src_hbm, dst_hbm, agg_out, src_v, dst_v, rows_v,
                 zbuf, agg_sh, sem):
    cid = lax.axis_index("c")
    sid = lax.axis_index("s")
    wid = cid * NS + sid

    zeros16 = jnp.zeros((16,), jnp.float32)
    for r in range(16):
        for j in range(DA // 16):
            zbuf[r, pl.ds(j * 16, 16)] = zeros16

    base_row = sid * ROWS_PT
    for t in range(ROWS_PT // 16):
        pltpu.sync_copy(zbuf, agg_sh.at[pl.ds(base_row + t * 16, 16)])
    plsc.subcore_barrier()

    def step(i, carry):
        base = (wid * CPW + i) * CHUNK
        pltpu.sync_copy(src_hbm.at[pl.ds(base, CHUNK)], src_v)
        pltpu.sync_copy(dst_hbm.at[pl.ds(base, CHUNK)], dst_v)
        pltpu.async_copy(x_hbm.at[src_v], rows_v, sem).wait()
        pltpu.sync_copy(rows_v, agg_sh.at[dst_v], add=True)
        return carry

    lax.fori_loop(0, CPW, step, 0)
    plsc.subcore_barrier()

    for t in range(ROWS_PT // CHUNK):
        r0 = base_row + t * CHUNK
        pltpu.sync_copy(agg_sh.at[pl.ds(r0, CHUNK)], rows_v)
        pltpu.sync_copy(rows_v, agg_out.at[cid, pl.ds(r0, CHUNK)])


@functools.cache
def _get_sc_agg():
    return pl.kernel(
        _sc_agg_body,
        out_type=jax.ShapeDtypeStruct((NC, NPAD, DA), jnp.float32),
        mesh=plsc.VectorSubcoreMesh(core_axis_name="c", subcore_axis_name="s",
                                    num_cores=NC, num_subcores=NS),
        scratch_types=[
            pltpu.VMEM((CHUNK,), jnp.int32),
            pltpu.VMEM((CHUNK,), jnp.int32),
            pltpu.VMEM((CHUNK, DA), jnp.float32),
            pltpu.VMEM((16, DA), jnp.float32),
            pltpu.VMEM_SHARED((NPAD, DA), jnp.float32),
            pltpu.SemaphoreType.DMA,
        ],
        compiler_params=pltpu.CompilerParams(use_tc_tiling_on_sc=False),
    )


def _tc_body(agg0, agg1, x_ref, wl, wr, b, o_ref, *, relu):
    agg = agg0[...] + agg1[...]
    deg = jnp.maximum(agg[:, D:D + 1], 1.0)
    m = agg / deg
    acc = (jnp.dot(m, wl[...], preferred_element_type=jnp.float32)
           + jnp.dot(x_ref[...], wr[...], preferred_element_type=jnp.float32)
           + b[...])
    if relu:
        acc = jnp.maximum(acc, 0.0)
    o_ref[:, pl.ds(0, D)] = acc
    if o_ref.shape[1] == DA:
        R = o_ref.shape[0]
        aug = jnp.where(
            lax.broadcasted_iota(jnp.int32, (R, DA - D), 1) == 0, 1.0, 0.0)
        o_ref[:, pl.ds(D, DA - D)] = aug


def _tc_dense(agg0, agg1, x, wlT, wrT, b, relu, aug_out):
    R = 2000
    OW = DA if aug_out else D
    spec_ag = pl.BlockSpec((R, DA), lambda i: (i, 0))
    spec_w = pl.BlockSpec((DA, D), lambda i: (0, 0))
    spec_b = pl.BlockSpec((1, D), lambda i: (0, 0))
    return pl.pallas_call(
        functools.partial(_tc_body, relu=relu),
        grid=(N // R,),
        in_specs=[spec_ag, spec_ag, spec_ag, spec_w, spec_w, spec_b],
        out_specs=pl.BlockSpec((R, OW), lambda i: (i, 0)),
        out_shape=jax.ShapeDtypeStruct((N, OW), jnp.float32),
    )(agg0, agg1, x, wlT, wrT, b)


def _pad_w(wT):
    return jnp.zeros((DA, D), jnp.float32).at[:D].set(wT)


def kernel(x, edge_index, W_l1, b_l1, W_r1, W_l2, b_l2, W_r2):
    src = edge_index[0]
    dst = edge_index[1]
    pad = EPAD - E
    srcp = jnp.concatenate([src, jnp.zeros((pad,), jnp.int32)])
    dstp = jnp.concatenate([dst, jnp.full((pad,), NPAD - 1, jnp.int32)])
    aug = jnp.tile(jnp.asarray([[1.0] + [0.0] * (DA - D - 1)], jnp.float32),
                   (N, 1))
    xa = jnp.concatenate([x, aug], axis=1)

    agg1 = _get_sc_agg()(xa, srcp, dstp)
    ha = _tc_dense(agg1[0], agg1[1], xa, _pad_w(W_l1.T), _pad_w(W_r1.T),
                   b_l1.reshape(1, D), relu=True, aug_out=True)
    agg2 = _get_sc_agg()(ha, srcp, dstp)
    out = _tc_dense(agg2[0], agg2[1], ha, _pad_w(W_l2.T), _pad_w(W_r2.T),
                    b_l2.reshape(1, D), relu=False, aug_out=False)
    return out
